# baseline (speedup 1.0000x reference)
import functools

import jax
import jax.numpy as jnp
from jax import lax
from jax.experimental import pallas as pl
from jax.experimental.pallas import tpu as pltpu
from jax.experimental.pallas import tpu_sc as plsc

N, D, K = 18432, 64, 1024
NCH = 2
CH_N = N // NCH
BM = 2304

_SC_INFO = plsc.get_sparse_core_info()
_NW = _SC_INFO.num_cores * _SC_INFO.num_subcores
_ROWS_PER_W = CH_N // _NW
_CHUNK = 96
_NCHUNK = _ROWS_PER_W // _CHUNK
_DP = 128


def _assign_body(z_ref, c_ref, codes_ref):
    z = z_ref[...]
    c = c_ref[...]
    d = lax.dot_general(c, z * (-2.0), (((1,), (1,)), ((), ())),
                        preferred_element_type=jnp.float32)
    d = d + jnp.sum(z * z, axis=1)[None, :]
    d = d + jnp.sum(c * c, axis=1)[:, None]
    codes_ref[0, 0, :] = jnp.argmin(d, axis=0).astype(jnp.int32)


def _tc_codes(z_chunk, centers):
    grid = CH_N // BM
    codes3 = pl.pallas_call(
        _assign_body,
        grid=(grid,),
        in_specs=[
            pl.BlockSpec((BM, D), lambda i: (i, 0)),
            pl.BlockSpec((K, D), lambda i: (0, 0)),
        ],
        out_specs=pl.BlockSpec((1, 1, BM), lambda i: (i, 0, 0)),
        out_shape=jax.ShapeDtypeStruct((grid, 1, BM), jnp.int32),
    )(z_chunk, centers)
    return codes3.reshape(CH_N)


_sc_mesh = plsc.VectorSubcoreMesh(core_axis_name="c", subcore_axis_name="s")


@functools.partial(
    pl.kernel,
    mesh=_sc_mesh,
    out_type=jax.ShapeDtypeStruct((CH_N, _DP), jnp.float32),
    scratch_types=[
        pltpu.VMEM((_ROWS_PER_W,), jnp.int32),
        pltpu.VMEM((_NCHUNK, _CHUNK, _DP), jnp.float32),
        pltpu.SemaphoreType.DMA,
        pltpu.SemaphoreType.DMA,
    ],
)
def _sc_gather(codes_hbm, centers_hbm, out_hbm, idx_v, rows_v, gsem, wsem):
    wid = lax.axis_index("s") * _SC_INFO.num_cores + lax.axis_index("c")
    base = wid * _ROWS_PER_W
    pltpu.sync_copy(codes_hbm.at[pl.ds(base, _ROWS_PER_W)], idx_v)
    gathers = [
        pltpu.async_copy(centers_hbm.at[idx_v.at[pl.ds(j * _CHUNK, _CHUNK)]],
                         rows_v.at[j], gsem)
        for j in range(_NCHUNK)
    ]
    writes = []
    for j in range(_NCHUNK):
        gathers[j].wait()
        writes.append(
            pltpu.async_copy(rows_v.at[j],
                             out_hbm.at[pl.ds(base + j * _CHUNK, _CHUNK)],
                             wsem))
    for w in writes:
        w.wait()


@jax.jit
def kernel(z, centers):
    centers_p = jnp.pad(centers, ((0, 0), (0, _DP - D)))
    codes_parts, quant_parts = [], []
    for i in range(NCH):
        codes_i = _tc_codes(lax.slice_in_dim(z, i * CH_N, (i + 1) * CH_N),
                            centers)
        codes_parts.append(codes_i)
        quant_parts.append(_sc_gather(codes_i, centers_p))
    codes = jnp.concatenate(codes_parts)
    quantized = jnp.concatenate(quant_parts)[:, :D]
    return codes, quantized


# speedup vs baseline: 1.5919x; 1.5919x over previous
import functools

import jax
import jax.numpy as jnp
from jax import lax
from jax.experimental import pallas as pl
from jax.experimental.pallas import tpu as pltpu
from jax.experimental.pallas import tpu_sc as plsc

N, D, K = 18432, 64, 1024
BM = 2304

_SC_INFO = plsc.get_sparse_core_info()
_NW = _SC_INFO.num_cores * _SC_INFO.num_subcores
_ROWS_PER_W = N // _NW
_CHUNK = 96
_NCHUNK = _ROWS_PER_W // _CHUNK


def _assign_quant_body(z_ref, c_ref, codes_ref, q_ref):
    z = z_ref[...]
    c = c_ref[...]
    d = lax.dot_general(c, z * (-2.0), (((1,), (1,)), ((), ())),
                        preferred_element_type=jnp.float32)
    d = d + jnp.sum(z * z, axis=1)[None, :]
    d = d + jnp.sum(c * c, axis=1)[:, None]
    codes = jnp.argmin(d, axis=0).astype(jnp.int32)
    codes_ref[0, 0, :] = codes
    onehot = (codes[:, None] == lax.broadcasted_iota(jnp.int32, (BM, K), 1))
    q_ref[...] = lax.dot_general(onehot.astype(jnp.float32), c,
                                 (((1,), (0,)), ((), ())),
                                 preferred_element_type=jnp.float32)


def _assign_body(z_ref, c_ref, codes_ref):
    z = z_ref[...]
    c = c_ref[...]
    d = lax.dot_general(c, z * (-2.0), (((1,), (1,)), ((), ())),
                        preferred_element_type=jnp.float32)
    d = d + jnp.sum(z * z, axis=1)[None, :]
    d = d + jnp.sum(c * c, axis=1)[:, None]
    codes_ref[0, 0, :] = jnp.argmin(d, axis=0).astype(jnp.int32)


def _tc_codes(z, centers):
    grid = N // BM
    codes3 = pl.pallas_call(
        _assign_body,
        grid=(grid,),
        in_specs=[
            pl.BlockSpec((BM, D), lambda i: (i, 0)),
            pl.BlockSpec((K, D), lambda i: (0, 0)),
        ],
        out_specs=pl.BlockSpec((1, 1, BM), lambda i: (i, 0, 0)),
        out_shape=jax.ShapeDtypeStruct((grid, 1, BM), jnp.int32),
    )(z, centers)
    return codes3.reshape(N)


_sc_mesh = plsc.VectorSubcoreMesh(core_axis_name="c", subcore_axis_name="s")
_DP = 128


@functools.partial(
    pl.kernel,
    mesh=_sc_mesh,
    out_type=jax.ShapeDtypeStruct((N, _DP), jnp.float32),
    scratch_types=[
        pltpu.VMEM((_NCHUNK, _CHUNK), jnp.int32),
        pltpu.VMEM((_NCHUNK, _CHUNK, _DP), jnp.float32),
        pltpu.SemaphoreType.DMA,
    ],
)
def _sc_gather(codes_hbm, centers_hbm, out_hbm, idx_v, rows_v, sem):
    wid = lax.axis_index("s") * _SC_INFO.num_cores + lax.axis_index("c")
    base = wid * _ROWS_PER_W
    copies = []
    for j in range(_NCHUNK):
        pltpu.sync_copy(codes_hbm.at[pl.ds(base + j * _CHUNK, _CHUNK)],
                        idx_v.at[j])
        copies.append(
            pltpu.async_copy(centers_hbm.at[idx_v.at[j]], rows_v.at[j], sem))
    for j in range(_NCHUNK):
        copies[j].wait()
        pltpu.sync_copy(rows_v.at[j],
                        out_hbm.at[pl.ds(base + j * _CHUNK, _CHUNK)])


@jax.jit
def kernel(z, centers):
    grid = N // BM
    codes3, quant = pl.pallas_call(
        _assign_quant_body,
        grid=(grid,),
        in_specs=[
            pl.BlockSpec((BM, D), lambda i: (i, 0)),
            pl.BlockSpec((K, D), lambda i: (0, 0)),
        ],
        out_specs=[
            pl.BlockSpec((1, 1, BM), lambda i: (i, 0, 0)),
            pl.BlockSpec((BM, D), lambda i: (i, 0)),
        ],
        out_shape=[
            jax.ShapeDtypeStruct((grid, 1, BM), jnp.int32),
            jax.ShapeDtypeStruct((N, D), jnp.float32),
        ],
    )(z, centers)
    return codes3.reshape(N), quant


# speedup vs baseline: 1.6863x; 1.0593x over previous
import functools

import jax
import jax.numpy as jnp
from jax import lax
from jax.experimental import pallas as pl
from jax.experimental.pallas import tpu as pltpu
from jax.experimental.pallas import tpu_sc as plsc

N, D, K = 18432, 64, 1024
BM = 2048

_SC_INFO = plsc.get_sparse_core_info()
_NW = _SC_INFO.num_cores * _SC_INFO.num_subcores
_ROWS_PER_W = N // _NW
_CHUNK = 96
_NCHUNK = _ROWS_PER_W // _CHUNK


def _assign_quant_body(z_ref, c_ref, codes_ref, q_ref):
    z = z_ref[...]
    c = c_ref[...]
    d = lax.dot_general(c, z * (-2.0), (((1,), (1,)), ((), ())),
                        preferred_element_type=jnp.float32)
    d = d + jnp.sum(z * z, axis=1)[None, :]
    d = d + jnp.sum(c * c, axis=1)[:, None]
    codes = jnp.argmin(d, axis=0).astype(jnp.int32)
    codes_ref[0, 0, :] = codes
    onehot = (codes[:, None] == lax.broadcasted_iota(jnp.int32, (BM, K), 1))
    q_ref[...] = lax.dot_general(onehot.astype(jnp.bfloat16),
                                 c.astype(jnp.bfloat16),
                                 (((1,), (0,)), ((), ())),
                                 preferred_element_type=jnp.float32)


def _assign_body(z_ref, c_ref, codes_ref):
    z = z_ref[...]
    c = c_ref[...]
    d = lax.dot_general(c, z * (-2.0), (((1,), (1,)), ((), ())),
                        preferred_element_type=jnp.float32)
    d = d + jnp.sum(z * z, axis=1)[None, :]
    d = d + jnp.sum(c * c, axis=1)[:, None]
    codes_ref[0, 0, :] = jnp.argmin(d, axis=0).astype(jnp.int32)


def _tc_codes(z, centers):
    grid = N // BM
    codes3 = pl.pallas_call(
        _assign_body,
        grid=(grid,),
        in_specs=[
            pl.BlockSpec((BM, D), lambda i: (i, 0)),
            pl.BlockSpec((K, D), lambda i: (0, 0)),
        ],
        out_specs=pl.BlockSpec((1, 1, BM), lambda i: (i, 0, 0)),
        out_shape=jax.ShapeDtypeStruct((grid, 1, BM), jnp.int32),
    )(z, centers)
    return codes3.reshape(N)


_sc_mesh = plsc.VectorSubcoreMesh(core_axis_name="c", subcore_axis_name="s")
_DP = 128


@functools.partial(
    pl.kernel,
    mesh=_sc_mesh,
    out_type=jax.ShapeDtypeStruct((N, _DP), jnp.float32),
    scratch_types=[
        pltpu.VMEM((_NCHUNK, _CHUNK), jnp.int32),
        pltpu.VMEM((_NCHUNK, _CHUNK, _DP), jnp.float32),
        pltpu.SemaphoreType.DMA,
    ],
)
def _sc_gather(codes_hbm, centers_hbm, out_hbm, idx_v, rows_v, sem):
    wid = lax.axis_index("s") * _SC_INFO.num_cores + lax.axis_index("c")
    base = wid * _ROWS_PER_W
    copies = []
    for j in range(_NCHUNK):
        pltpu.sync_copy(codes_hbm.at[pl.ds(base + j * _CHUNK, _CHUNK)],
                        idx_v.at[j])
        copies.append(
            pltpu.async_copy(centers_hbm.at[idx_v.at[j]], rows_v.at[j], sem))
    for j in range(_NCHUNK):
        copies[j].wait()
        pltpu.sync_copy(rows_v.at[j],
                        out_hbm.at[pl.ds(base + j * _CHUNK, _CHUNK)])


@jax.jit
def kernel(z, centers):
    grid = N // BM
    codes3, quant = pl.pallas_call(
        _assign_quant_body,
        grid=(grid,),
        in_specs=[
            pl.BlockSpec((BM, D), lambda i: (i, 0)),
            pl.BlockSpec((K, D), lambda i: (0, 0)),
        ],
        out_specs=[
            pl.BlockSpec((1, 1, BM), lambda i: (i, 0, 0)),
            pl.BlockSpec((BM, D), lambda i: (i, 0)),
        ],
        out_shape=[
            jax.ShapeDtypeStruct((grid, 1, BM), jnp.int32),
            jax.ShapeDtypeStruct((N, D), jnp.float32),
        ],
    )(z, centers)
    return codes3.reshape(N), quant


# speedup vs baseline: 1.6987x; 1.0074x over previous
import jax
import jax.numpy as jnp
from jax import lax
from jax.experimental import pallas as pl

N, D, K = 18432, 64, 1024
BM = 2048


def _assign_quant_body(z_ref, c_ref, codes_ref, q_ref):
    z = z_ref[...]
    c = c_ref[...]
    d = lax.dot_general(c, z * (-2.0), (((1,), (1,)), ((), ())),
                        preferred_element_type=jnp.float32)
    d = d + jnp.sum(z * z, axis=1)[None, :]
    d = d + jnp.sum(c * c, axis=1)[:, None]
    codes = jnp.argmin(d, axis=0).astype(jnp.int32)
    codes_ref[0, 0, :] = codes
    onehot = (codes[:, None] == lax.broadcasted_iota(jnp.int32, (BM, K), 1))
    q_ref[...] = lax.dot_general(onehot.astype(jnp.float32), c,
                                 (((1,), (0,)), ((), ())),
                                 preferred_element_type=jnp.float32)


@jax.jit
def kernel(z, centers):
    grid = N // BM
    codes3, quant = pl.pallas_call(
        _assign_quant_body,
        grid=(grid,),
        in_specs=[
            pl.BlockSpec((BM, D), lambda i: (i, 0)),
            pl.BlockSpec((K, D), lambda i: (0, 0)),
        ],
        out_specs=[
            pl.BlockSpec((1, 1, BM), lambda i: (i, 0, 0)),
            pl.BlockSpec((BM, D), lambda i: (i, 0)),
        ],
        out_shape=[
            jax.ShapeDtypeStruct((grid, 1, BM), jnp.int32),
            jax.ShapeDtypeStruct((N, D), jnp.float32),
        ],
    )(z, centers)
    return codes3.reshape(N), quant
